# Initial kernel scaffold; baseline (speedup 1.0000x reference)
import jax
import jax.numpy as jnp
from jax.experimental import pallas as pl


def kernel(input_ids, token_table, pos_table):
    raise NotImplementedError("write your pallas kernel here")


# speedup vs baseline: 1.0039x; 1.0039x over previous
import functools

import jax
import jax.numpy as jnp
from jax import lax
from jax.experimental import pallas as pl
from jax.experimental.pallas import tpu as pltpu
from jax.experimental.pallas import tpu_sc as plsc

NC, NS, L = 2, 16, 16
NW = NC * NS
B, S, D = 4, 2048, 128
T = B * S
TPW = T // NW
CH = 128
NCH = TPW // CH


def _body(ids_hbm, tok_hbm, pos_hbm, out_hbm, idx_v, rows_v, pos_v, sem):
    wid = lax.axis_index("s") * NC + lax.axis_index("c")
    base = wid * TPW
    pos_base = lax.rem(base, S)

    pltpu.sync_copy(ids_hbm.at[pl.ds(base, TPW)], idx_v)
    copies = [
        pltpu.async_copy(
            tok_hbm.at[idx_v.at[pl.ds(j * CH, CH)]],
            rows_v.at[pl.ds(j * CH, CH)],
            sem,
        )
        for j in range(NCH)
    ]
    pltpu.sync_copy(pos_hbm.at[pl.ds(pos_base, TPW)], pos_v)
    for c in copies:
        c.wait()

    def row_add(r, carry):
        for c in range(D // L):
            sl = pl.ds(c * L, L)
            rows_v[r, sl] = rows_v[r, sl] + pos_v[r, sl]
        return carry

    lax.fori_loop(0, TPW, row_add, 0, unroll=2)

    pltpu.sync_copy(rows_v, out_hbm.at[pl.ds(base, TPW)])


@jax.jit
def _embed(ids_flat, tok, pos):
    mesh = plsc.VectorSubcoreMesh(
        core_axis_name="c", subcore_axis_name="s", num_cores=NC, num_subcores=NS
    )
    return pl.kernel(
        _body,
        out_type=jax.ShapeDtypeStruct((T, D), jnp.float32),
        mesh=mesh,
        scratch_types=[
            pltpu.VMEM((TPW,), jnp.int32),
            pltpu.VMEM((TPW, D), jnp.float32),
            pltpu.VMEM((TPW, D), jnp.float32),
            pltpu.SemaphoreType.DMA,
        ],
    )(ids_flat, tok, pos)


def kernel(input_ids, token_table, pos_table):
    ids_flat = input_ids.reshape(T).astype(jnp.int32)
    out = _embed(ids_flat, token_table, pos_table)
    return out.reshape(B, S, D)


# speedup vs baseline: 1.3421x; 1.3369x over previous
import functools

import jax
import jax.numpy as jnp
from jax import lax
from jax.experimental import pallas as pl
from jax.experimental.pallas import tpu as pltpu
from jax.experimental.pallas import tpu_sc as plsc

NC, NS, L = 2, 16, 16
NW = NC * NS
B, S, D = 4, 2048, 128
T = B * S
TPW = T // NW
CH = 128
NCH = TPW // CH


def _body(ids_hbm, tok_hbm, pos_hbm, out_hbm, idx_v, rows_v, sem):
    wid = lax.axis_index("s") * NC + lax.axis_index("c")
    base = wid * TPW
    pos_base = lax.rem(base, S)

    pltpu.sync_copy(ids_hbm.at[pl.ds(base, TPW)], idx_v)
    pltpu.sync_copy(pos_hbm.at[pl.ds(pos_base, TPW)], rows_v)
    copies = [
        pltpu.async_copy(
            tok_hbm.at[idx_v.at[pl.ds(j * CH, CH)]],
            rows_v.at[pl.ds(j * CH, CH)],
            sem,
            add=True,
        )
        for j in range(NCH)
    ]
    for c in copies:
        c.wait()

    pltpu.sync_copy(rows_v, out_hbm.at[pl.ds(base, TPW)])


@jax.jit
def _embed(ids_flat, tok, pos):
    mesh = plsc.VectorSubcoreMesh(
        core_axis_name="c", subcore_axis_name="s", num_cores=NC, num_subcores=NS
    )
    return pl.kernel(
        _body,
        out_type=jax.ShapeDtypeStruct((T, D), jnp.float32),
        mesh=mesh,
        scratch_types=[
            pltpu.VMEM((TPW,), jnp.int32),
            pltpu.VMEM((TPW, D), jnp.float32),
            pltpu.SemaphoreType.DMA,
        ],
    )(ids_flat, tok, pos)


def kernel(input_ids, token_table, pos_table):
    ids_flat = input_ids.reshape(T).astype(jnp.int32)
    out = _embed(ids_flat, token_table, pos_table)
    return out.reshape(B, S, D)


# speedup vs baseline: 1.3825x; 1.0301x over previous
import jax
import jax.numpy as jnp
from jax import lax
from jax.experimental import pallas as pl
from jax.experimental.pallas import tpu as pltpu
from jax.experimental.pallas import tpu_sc as plsc

NC, NS, L = 2, 16, 16
NW = NC * NS
B, S, D = 4, 2048, 128
T = B * S
TPW = T // NW
CH = 128
NCH = TPW // CH


def _body(ids_hbm, tok_hbm, pos_hbm, out_hbm, idx_v, rows_v,
          s_idx, s_p0, s_p1, s_g0, s_g1, s_out):
    wid = lax.axis_index("s") * NC + lax.axis_index("c")
    base = wid * TPW
    pos_base = lax.rem(base, S)
    s_p = (s_p0, s_p1)
    s_g = (s_g0, s_g1)

    c_idx = pltpu.async_copy(ids_hbm.at[pl.ds(base, TPW)], idx_v, s_idx)
    c_pos = [
        pltpu.async_copy(
            pos_hbm.at[pl.ds(pos_base + j * CH, CH)],
            rows_v.at[pl.ds(j * CH, CH)],
            s_p[j],
        )
        for j in range(NCH)
    ]
    c_idx.wait()
    gathers = []
    for j in range(NCH):
        c_pos[j].wait()
        gathers.append(
            pltpu.async_copy(
                tok_hbm.at[idx_v.at[pl.ds(j * CH, CH)]],
                rows_v.at[pl.ds(j * CH, CH)],
                s_g[j],
                add=True,
            )
        )
    outs = []
    for j in range(NCH):
        gathers[j].wait()
        outs.append(
            pltpu.async_copy(
                rows_v.at[pl.ds(j * CH, CH)],
                out_hbm.at[pl.ds(base + j * CH, CH)],
                s_out,
            )
        )
    for c in outs:
        c.wait()


@jax.jit
def _embed(ids_flat, tok, pos):
    mesh = plsc.VectorSubcoreMesh(
        core_axis_name="c", subcore_axis_name="s", num_cores=NC, num_subcores=NS
    )
    return pl.kernel(
        _body,
        out_type=jax.ShapeDtypeStruct((T, D), jnp.float32),
        mesh=mesh,
        scratch_types=[
            pltpu.VMEM((TPW,), jnp.int32),
            pltpu.VMEM((TPW, D), jnp.float32),
            pltpu.SemaphoreType.DMA,
            pltpu.SemaphoreType.DMA,
            pltpu.SemaphoreType.DMA,
            pltpu.SemaphoreType.DMA,
            pltpu.SemaphoreType.DMA,
            pltpu.SemaphoreType.DMA,
        ],
    )(ids_flat, tok, pos)


def kernel(input_ids, token_table, pos_table):
    ids_flat = input_ids.reshape(T).astype(jnp.int32)
    out = _embed(ids_flat, token_table, pos_table)
    return out.reshape(B, S, D)
